# baseline (speedup 1.0000x reference)
import dataclasses
import functools

import jax
import jax.numpy as jnp
from jax import lax
from jax.experimental import pallas as pl
from jax.experimental.pallas import tpu as pltpu
from jax.experimental.pallas import tpu_sc as plsc

_NW = 32
_D = 64
_G = 4
_SPLITS = ((0, 104), (104, 96))


def _clip_scale(ss):
    i = plsc.bitcast(ss, jnp.int32)
    i = jnp.int32(0x5F3759DF) - (i >> 1)
    y = plsc.bitcast(i, jnp.float32)
    for _ in range(3):
        y = y * (1.5 - 0.5 * ss * y * y)
    return jnp.where(ss > 1.0, y, jnp.float32(1.0))


def kernel(uid, seq, pos, neg, user_embs, item_embs, item_beta, trans):
    B, L = seq.shape
    b_per_w = B // _NW
    nch = b_per_w // _G
    W = _G * L

    seqf = seq.reshape(-1)
    posf = pos.reshape(-1)
    negf = neg.reshape(-1)
    betaf = item_beta
    uid2 = uid.reshape(B // _G, _G)

    mesh = plsc.VectorSubcoreMesh(core_axis_name="c", subcore_axis_name="s")
    out_sds = jax.ShapeDtypeStruct((B * L,), jnp.float32)
    cp = pltpu.CompilerParams()
    for _f, _v in (("needs_layout_passes", False),
                   ("use_tc_tiling_on_sc", False)):
        if _f in pltpu.CompilerParams.__dataclass_fields__:
            cp = dataclasses.replace(cp, **{_f: _v})

    def _buf_set():
        return [
            pltpu.VMEM((W,), jnp.int32),
            pltpu.VMEM((W,), jnp.int32),
            pltpu.VMEM((W,), jnp.int32),
            pltpu.VMEM((_G, _D), jnp.float32),
            pltpu.VMEM((W, _D), jnp.float32),
            pltpu.VMEM((W, _D), jnp.float32),
            pltpu.VMEM((W, _D), jnp.float32),
            pltpu.VMEM((W, 1), jnp.float32),
            pltpu.VMEM((W, 1), jnp.float32),
            pltpu.VMEM((W,), jnp.float32),
            pltpu.VMEM((W,), jnp.float32),
            pltpu.SemaphoreType.DMA,
            pltpu.SemaphoreType.DMA,
            pltpu.SemaphoreType.DMA,
        ]

    @functools.partial(
        pl.kernel,
        mesh=mesh,
        compiler_params=cp,
        out_type=[out_sds, out_sds],
        scratch_types=[
            pltpu.VMEM((nch, _G), jnp.int32),
            pltpu.VMEM((_D,), jnp.float32),
            pltpu.VMEM((_G, _D), jnp.float32),
        ] + _buf_set() + _buf_set(),
    )
    def run(uid_hbm, seq_hbm, pos_hbm, neg_hbm, user_hbm, item_hbm, beta_hbm,
            trans_hbm, outp_hbm, outn_hbm, uid_v, trans_v, arows, *bufs):
        nb = len(_buf_set())
        sets = (bufs[:nb], bufs[nb:])
        wid = lax.axis_index("s") * 2 + lax.axis_index("c")
        tb = wid * b_per_w

        pltpu.sync_copy(uid_hbm.at[pl.ds(wid * nch, nch), :], uid_v)
        pltpu.sync_copy(trans_hbm, trans_v)

        def idx_copies(c, S):
            seqi, posi, negi = S[0], S[1], S[2]
            pb = tb * L + c * W
            return [
                (seq_hbm.at[pl.ds(pb, W)], seqi),
                (pos_hbm.at[pl.ds(pb, W)], posi),
                (neg_hbm.at[pl.ds(pb, W)], negi),
            ]

        def gather_copies(c, S):
            seqi, posi, negi, urows = S[0], S[1], S[2], S[3]
            srows, prows, nrows, pbeta, nbeta = S[4], S[5], S[6], S[7], S[8]
            out = [(user_hbm.at[uid_v.at[c]], urows)]
            for (o, n) in _SPLITS:
                sl = pl.ds(o, n)
                out += [
                    (item_hbm.at[seqi.at[sl]], srows.at[sl]),
                    (item_hbm.at[posi.at[sl]], prows.at[sl]),
                    (item_hbm.at[negi.at[sl]], nrows.at[sl]),
                    (beta_hbm.at[posi.at[sl]], pbeta.at[sl]),
                    (beta_hbm.at[negi.at[sl]], nbeta.at[sl]),
                ]
            return out

        def out_copies(c, S):
            pb = tb * L + c * W
            return [(S[9], outp_hbm.at[pl.ds(pb, W)]),
                    (S[10], outn_hbm.at[pl.ds(pb, W)])]

        def fire(pairs, sem):
            for s, d in pairs:
                pltpu.async_copy(s, d, sem)

        def drain(pairs, sem):
            for s, d in pairs:
                pltpu.make_async_copy(s, d, sem).wait()

        lanes = lax.iota(jnp.int32, 16)
        zeros = jnp.zeros((16,), jnp.float32)

        def compute(c, S):
            urows = S[3]
            srows, prows, nrows, pbeta, nbeta = S[4], S[5], S[6], S[7], S[8]
            outp_v, outn_v = S[9], S[10]

            ssu_vec = zeros
            for g in range(_G):
                acc = None
                for k in range(4):
                    u = urows[g, pl.ds(16 * k, 16)]
                    acc = u * u if acc is None else acc + u * u
                ssu_vec = jnp.where(lanes == g, jnp.sum(acc), ssu_vec)
            scu_vec = _clip_scale(ssu_vec)
            for g in range(_G):
                scu = scu_vec[g]
                for k in range(4):
                    arows[g, pl.ds(16 * k, 16)] = (
                        urows[g, pl.ds(16 * k, 16)] * scu
                        + trans_v[pl.ds(16 * k, 16)])

            def _emit_group(rows, a_of, A_of, store):
                SSs = SSp = SSn = ASv = APv = ANv = SPv = SNv = zeros
                Av = zeros
                for j, r in enumerate(rows):
                    s = [srows[r, pl.ds(16 * k, 16)] for k in range(4)]
                    p = [prows[r, pl.ds(16 * k, 16)] for k in range(4)]
                    n = [nrows[r, pl.ds(16 * k, 16)] for k in range(4)]
                    a = a_of(j)
                    m = lanes == j

                    def dot(x, y):
                        acc = x[0] * y[0]
                        for k in range(1, 4):
                            acc = acc + x[k] * y[k]
                        return jnp.sum(acc)

                    SSs = jnp.where(m, dot(s, s), SSs)
                    SSp = jnp.where(m, dot(p, p), SSp)
                    SSn = jnp.where(m, dot(n, n), SSn)
                    ASv = jnp.where(m, dot(a, s), ASv)
                    APv = jnp.where(m, dot(a, p), APv)
                    ANv = jnp.where(m, dot(a, n), ANv)
                    SPv = jnp.where(m, dot(s, p), SPv)
                    SNv = jnp.where(m, dot(s, n), SNv)
                    if A_of is None:
                        Av = jnp.where(m, dot(a, a), Av)
                if A_of is not None:
                    Av = A_of
                al = _clip_scale(SSs)
                be = _clip_scale(SSp)
                ga = _clip_scale(SSn)
                base = Av + al * al * SSs + 2.0 * al * ASv
                distp = base + be * be * SSp - 2.0 * (be * APv + al * be * SPv)
                distn = base + ga * ga * SSn - 2.0 * (ga * ANv + al * ga * SNv)
                store(distp, distn)

            @pl.loop(0, _G)
            def _b(g):
                a = [arows[g, pl.ds(16 * k, 16)] for k in range(4)]
                accA = a[0] * a[0]
                for k in range(1, 4):
                    accA = accA + a[k] * a[k]
                A = jnp.sum(accA)

                @pl.loop(0, 3)
                def _tl(tl):
                    r0 = g * L + 16 * tl

                    def store(distp, distn):
                        rvec = lanes + r0
                        zv = lanes * 0
                        pb = plsc.load_gather(pbeta, [rvec, zv])
                        nb_ = plsc.load_gather(nbeta, [rvec, zv])
                        outp_v[pl.ds(r0, 16)] = pb - distp
                        outn_v[pl.ds(r0, 16)] = nb_ - distn

                    _emit_group([r0 + j for j in range(16)],
                                lambda j: a, A, store)

            lrows = [(j // 2) * L + 48 + (j % 2) if j < 2 * _G else 0
                     for j in range(16)]
            lmask = lanes < 2 * _G
            lidx = jnp.where(lmask, (lanes >> 1) * L + 48 + (lanes & 1), 0)

            def lstore(distp, distn):
                zv = lanes * 0
                pb = plsc.load_gather(pbeta, [lidx, zv])
                nb_ = plsc.load_gather(nbeta, [lidx, zv])
                plsc.store_scatter(outp_v, [lidx], pb - distp, mask=lmask)
                plsc.store_scatter(outn_v, [lidx], nb_ - distn, mask=lmask)

            _emit_group(lrows,
                        lambda j: [arows[(j // 2) % _G, pl.ds(16 * k, 16)]
                                   for k in range(4)],
                        None, lstore)

        for s, d in idx_copies(0, sets[0]):
            pltpu.async_copy(s, d, sets[0][11]).wait()
        fire(gather_copies(0, sets[0]), sets[0][12])
        fire(idx_copies(1, sets[1]), sets[1][11])

        @pl.loop(0, nch, step=2)
        def _body(c):
            s0, s1 = sets
            drain(idx_copies(c + 1, s1), s1[11])
            fire(gather_copies(c + 1, s1), s1[12])
            drain(gather_copies(c, s0), s0[12])
            @pl.when(c + 2 < nch)
            def _():
                fire(idx_copies(c + 2, s0), s0[11])
            @pl.when(c >= 2)
            def _():
                drain(out_copies(c - 2, s0), s0[13])
            compute(c, s0)
            fire(out_copies(c, s0), s0[13])
            @pl.when(c + 2 < nch)
            def _():
                drain(idx_copies(c + 2, s0), s0[11])
                fire(gather_copies(c + 2, s0), s0[12])
            drain(gather_copies(c + 1, s1), s1[12])
            @pl.when(c + 2 < nch)
            def _():
                fire(idx_copies(c + 3, s1), s1[11])
            @pl.when(c >= 2)
            def _():
                drain(out_copies(c - 1, s1), s1[13])
            compute(c + 1, s1)
            fire(out_copies(c + 1, s1), s1[13])

        drain(out_copies(nch - 2, sets[0]), sets[0][13])
        drain(out_copies(nch - 1, sets[1]), sets[1][13])

    outp, outn = run(uid2, seqf, posf, negf, user_embs, item_embs, betaf,
                     trans)
    return outp.reshape(B, L, 1), outn.reshape(B, L, 1)


# speedup vs baseline: 1.6329x; 1.6329x over previous
import dataclasses
import functools

import jax
import jax.numpy as jnp
from jax import lax
from jax.experimental import pallas as pl
from jax.experimental.pallas import tpu as pltpu
from jax.experimental.pallas import tpu_sc as plsc

_NW = 32
_D = 64
_G = 4
_SPLITS = ((0, 104), (104, 96))


def _clip_scale(ss):
    i = plsc.bitcast(ss, jnp.int32)
    i = jnp.int32(0x5F3759DF) - (i >> 1)
    y = plsc.bitcast(i, jnp.float32)
    for _ in range(3):
        y = y * (1.5 - 0.5 * ss * y * y)
    return jnp.where(ss > 1.0, y, jnp.float32(1.0))


def kernel(uid, seq, pos, neg, user_embs, item_embs, item_beta, trans):
    B, L = seq.shape
    b_per_w = B // _NW
    nch = b_per_w // _G
    W = _G * L

    seqf = seq.reshape(-1)
    posf = pos.reshape(-1)
    negf = neg.reshape(-1)
    betaf = item_beta.reshape(-1)
    uid2 = uid.reshape(B // _G, _G)

    mesh = plsc.VectorSubcoreMesh(core_axis_name="c", subcore_axis_name="s")
    out_sds = jax.ShapeDtypeStruct((B * L,), jnp.float32)
    cp = pltpu.CompilerParams()
    for _f, _v in (("needs_layout_passes", False),
                   ("use_tc_tiling_on_sc", False)):
        if _f in pltpu.CompilerParams.__dataclass_fields__:
            cp = dataclasses.replace(cp, **{_f: _v})

    def _buf_set():
        return [
            pltpu.VMEM((W,), jnp.int32),
            pltpu.VMEM((W,), jnp.int32),
            pltpu.VMEM((W,), jnp.int32),
            pltpu.VMEM((_G, _D), jnp.float32),
            pltpu.VMEM((W, _D), jnp.float32),
            pltpu.VMEM((W, _D), jnp.float32),
            pltpu.VMEM((W, _D), jnp.float32),
            pltpu.VMEM((W,), jnp.float32),
            pltpu.VMEM((W,), jnp.float32),
            pltpu.SemaphoreType.DMA,
            pltpu.SemaphoreType.DMA,
            pltpu.SemaphoreType.DMA,
        ]

    @functools.partial(
        pl.kernel,
        mesh=mesh,
        compiler_params=cp,
        out_type=[out_sds, out_sds],
        scratch_types=[
            pltpu.VMEM((nch, _G), jnp.int32),
            pltpu.VMEM((_D,), jnp.float32),
            pltpu.VMEM((_G, _D), jnp.float32),
        ] + _buf_set() + _buf_set(),
    )
    def run(uid_hbm, seq_hbm, pos_hbm, neg_hbm, user_hbm, item_hbm,
            trans_hbm, outp_hbm, outn_hbm, uid_v, trans_v, arows, *bufs):
        nb = len(_buf_set())
        sem_idx, sem_gat, sem_out = 9, 10, 11
        sets = (bufs[:nb], bufs[nb:])
        wid = lax.axis_index("s") * 2 + lax.axis_index("c")
        tb = wid * b_per_w

        pltpu.sync_copy(uid_hbm.at[pl.ds(wid * nch, nch), :], uid_v)
        pltpu.sync_copy(trans_hbm, trans_v)

        def idx_copies(c, S):
            seqi, posi, negi = S[0], S[1], S[2]
            pb = tb * L + c * W
            return [
                (seq_hbm.at[pl.ds(pb, W)], seqi),
                (pos_hbm.at[pl.ds(pb, W)], posi),
                (neg_hbm.at[pl.ds(pb, W)], negi),
            ]

        def gather_copies(c, S):
            seqi, posi, negi, urows = S[0], S[1], S[2], S[3]
            srows, prows, nrows = S[4], S[5], S[6]
            out = [(user_hbm.at[uid_v.at[c]], urows)]
            for (o, n) in _SPLITS:
                sl = pl.ds(o, n)
                out += [
                    (item_hbm.at[seqi.at[sl]], srows.at[sl]),
                    (item_hbm.at[posi.at[sl]], prows.at[sl]),
                    (item_hbm.at[negi.at[sl]], nrows.at[sl]),
                ]
            return out

        def out_copies(c, S):
            pb = tb * L + c * W
            return [(S[7], outp_hbm.at[pl.ds(pb, W)]),
                    (S[8], outn_hbm.at[pl.ds(pb, W)])]

        def fire(pairs, sem):
            for s, d in pairs:
                pltpu.async_copy(s, d, sem)

        def drain(pairs, sem):
            for s, d in pairs:
                pltpu.make_async_copy(s, d, sem).wait()

        lanes = lax.iota(jnp.int32, 16)
        zeros = jnp.zeros((16,), jnp.float32)

        def compute(c, S):
            urows = S[3]
            srows, prows, nrows = S[4], S[5], S[6]
            outp_v, outn_v = S[7], S[8]

            ssu_vec = zeros
            for g in range(_G):
                acc = None
                for k in range(4):
                    u = urows[g, pl.ds(16 * k, 16)]
                    acc = u * u if acc is None else acc + u * u
                ssu_vec = jnp.where(lanes == g, jnp.sum(acc), ssu_vec)
            scu_vec = _clip_scale(ssu_vec)
            for g in range(_G):
                scu = scu_vec[g]
                for k in range(4):
                    arows[g, pl.ds(16 * k, 16)] = (
                        urows[g, pl.ds(16 * k, 16)] * scu
                        + trans_v[pl.ds(16 * k, 16)])

            def _emit_group(rows, a_of, A_of, store):
                SSs = SSp = SSn = ASv = APv = ANv = SPv = SNv = zeros
                Av = zeros
                for j, r in enumerate(rows):
                    s = [srows[r, pl.ds(16 * k, 16)] for k in range(4)]
                    p = [prows[r, pl.ds(16 * k, 16)] for k in range(4)]
                    n = [nrows[r, pl.ds(16 * k, 16)] for k in range(4)]
                    a = a_of(j)
                    m = lanes == j

                    def dot(x, y):
                        acc = x[0] * y[0]
                        for k in range(1, 4):
                            acc = acc + x[k] * y[k]
                        return jnp.sum(acc)

                    SSs = jnp.where(m, dot(s, s), SSs)
                    SSp = jnp.where(m, dot(p, p), SSp)
                    SSn = jnp.where(m, dot(n, n), SSn)
                    ASv = jnp.where(m, dot(a, s), ASv)
                    APv = jnp.where(m, dot(a, p), APv)
                    ANv = jnp.where(m, dot(a, n), ANv)
                    SPv = jnp.where(m, dot(s, p), SPv)
                    SNv = jnp.where(m, dot(s, n), SNv)
                    if A_of is None:
                        Av = jnp.where(m, dot(a, a), Av)
                if A_of is not None:
                    Av = A_of
                al = _clip_scale(SSs)
                be = _clip_scale(SSp)
                ga = _clip_scale(SSn)
                base = Av + al * al * SSs + 2.0 * al * ASv
                distp = base + be * be * SSp - 2.0 * (be * APv + al * be * SPv)
                distn = base + ga * ga * SSn - 2.0 * (ga * ANv + al * ga * SNv)
                store(distp, distn)

            @pl.loop(0, _G)
            def _b(g):
                a = [arows[g, pl.ds(16 * k, 16)] for k in range(4)]
                accA = a[0] * a[0]
                for k in range(1, 4):
                    accA = accA + a[k] * a[k]
                A = jnp.sum(accA)

                @pl.loop(0, 3)
                def _tl(tl):
                    r0 = g * L + 16 * tl

                    def store(distp, distn):
                        outp_v[pl.ds(r0, 16)] = distp
                        outn_v[pl.ds(r0, 16)] = distn

                    _emit_group([r0 + j for j in range(16)],
                                lambda j: a, A, store)

            lrows = [(j // 2) * L + 48 + (j % 2) if j < 2 * _G else 0
                     for j in range(16)]
            lmask = lanes < 2 * _G
            lidx = jnp.where(lmask, (lanes >> 1) * L + 48 + (lanes & 1), 0)

            def lstore(distp, distn):
                plsc.store_scatter(outp_v, [lidx], distp, mask=lmask)
                plsc.store_scatter(outn_v, [lidx], distn, mask=lmask)

            _emit_group(lrows,
                        lambda j: [arows[(j // 2) % _G, pl.ds(16 * k, 16)]
                                   for k in range(4)],
                        None, lstore)

        for s, d in idx_copies(0, sets[0]):
            pltpu.async_copy(s, d, sets[0][sem_idx]).wait()
        fire(gather_copies(0, sets[0]), sets[0][sem_gat])
        fire(idx_copies(1, sets[1]), sets[1][sem_idx])

        @pl.loop(0, nch, step=2)
        def _body(c):
            s0, s1 = sets
            drain(idx_copies(c + 1, s1), s1[sem_idx])
            fire(gather_copies(c + 1, s1), s1[sem_gat])
            drain(gather_copies(c, s0), s0[sem_gat])
            @pl.when(c + 2 < nch)
            def _():
                fire(idx_copies(c + 2, s0), s0[sem_idx])
            @pl.when(c >= 2)
            def _():
                drain(out_copies(c - 2, s0), s0[sem_out])
            compute(c, s0)
            fire(out_copies(c, s0), s0[sem_out])
            @pl.when(c + 2 < nch)
            def _():
                drain(idx_copies(c + 2, s0), s0[sem_idx])
                fire(gather_copies(c + 2, s0), s0[sem_gat])
            drain(gather_copies(c + 1, s1), s1[sem_gat])
            @pl.when(c + 2 < nch)
            def _():
                fire(idx_copies(c + 3, s1), s1[sem_idx])
            @pl.when(c >= 2)
            def _():
                drain(out_copies(c - 1, s1), s1[sem_out])
            compute(c + 1, s1)
            fire(out_copies(c + 1, s1), s1[sem_out])

        drain(out_copies(nch - 2, sets[0]), sets[0][sem_out])
        drain(out_copies(nch - 1, sets[1]), sets[1][sem_out])

    dp, dn = run(uid2, seqf, posf, negf, user_embs, item_embs, trans)

    CW = 800
    nch2 = (B * L // _NW) // CW
    sp2 = tuple((o, 128 if o + 128 <= CW else CW - o)
                for o in range(0, CW, 128))

    def _buf_set2():
        return [
            pltpu.VMEM((CW,), jnp.int32),
            pltpu.VMEM((CW,), jnp.int32),
            pltpu.VMEM((CW,), jnp.float32),
            pltpu.VMEM((CW,), jnp.float32),
            pltpu.VMEM((CW,), jnp.float32),
            pltpu.VMEM((CW,), jnp.float32),
            pltpu.VMEM((CW,), jnp.float32),
            pltpu.VMEM((CW,), jnp.float32),
            pltpu.SemaphoreType.DMA,
            pltpu.SemaphoreType.DMA,
            pltpu.SemaphoreType.DMA,
        ]

    @functools.partial(
        pl.kernel,
        mesh=mesh,
        compiler_params=cp,
        out_type=[out_sds, out_sds],
        scratch_types=_buf_set2() + _buf_set2(),
    )
    def run2(pos_hbm, neg_hbm, beta_hbm, dp_hbm, dn_hbm,
             outp_hbm, outn_hbm, *bufs):
        nb = len(_buf_set2())
        sem_in, sem_gat, sem_out = 8, 9, 10
        sets = (bufs[:nb], bufs[nb:])
        wid = lax.axis_index("s") * 2 + lax.axis_index("c")
        base = wid * (B * L // _NW)

        def in_copies(c, S):
            pb = base + c * CW
            return [
                (pos_hbm.at[pl.ds(pb, CW)], S[0]),
                (neg_hbm.at[pl.ds(pb, CW)], S[1]),
                (dp_hbm.at[pl.ds(pb, CW)], S[2]),
                (dn_hbm.at[pl.ds(pb, CW)], S[3]),
            ]

        def gather_copies2(c, S):
            out = []
            for (o, n) in sp2:
                sl = pl.ds(o, n)
                out += [(beta_hbm.at[S[0].at[sl]], S[4].at[sl]),
                        (beta_hbm.at[S[1].at[sl]], S[5].at[sl])]
            return out

        def out_copies2(c, S):
            pb = base + c * CW
            return [(S[6], outp_hbm.at[pl.ds(pb, CW)]),
                    (S[7], outn_hbm.at[pl.ds(pb, CW)])]

        def fire(pairs, sem):
            for s, d in pairs:
                pltpu.async_copy(s, d, sem)

        def drain(pairs, sem):
            for s, d in pairs:
                pltpu.make_async_copy(s, d, sem).wait()

        def compute2(c, S):
            @pl.loop(0, CW // 16)
            def _t(t):
                sl = pl.ds(t * 16, 16)
                S[6][sl] = S[4][sl] - S[2][sl]
                S[7][sl] = S[5][sl] - S[3][sl]

        for s, d in in_copies(0, sets[0]):
            pltpu.async_copy(s, d, sets[0][sem_in]).wait()
        fire(gather_copies2(0, sets[0]), sets[0][sem_gat])
        fire(in_copies(1, sets[1]), sets[1][sem_in])

        @pl.loop(0, nch2, step=2)
        def _body(c):
            s0, s1 = sets
            drain(in_copies(c + 1, s1), s1[sem_in])
            fire(gather_copies2(c + 1, s1), s1[sem_gat])
            drain(gather_copies2(c, s0), s0[sem_gat])
            @pl.when(c + 2 < nch2)
            def _():
                fire(in_copies(c + 2, s0), s0[sem_in])
            @pl.when(c >= 2)
            def _():
                drain(out_copies2(c - 2, s0), s0[sem_out])
            compute2(c, s0)
            fire(out_copies2(c, s0), s0[sem_out])
            @pl.when(c + 2 < nch2)
            def _():
                drain(in_copies(c + 2, s0), s0[sem_in])
                fire(gather_copies2(c + 2, s0), s0[sem_gat])
            drain(gather_copies2(c + 1, s1), s1[sem_gat])
            @pl.when(c + 2 < nch2)
            def _():
                fire(in_copies(c + 3, s1), s1[sem_in])
            @pl.when(c >= 2)
            def _():
                drain(out_copies2(c - 1, s1), s1[sem_out])
            compute2(c + 1, s1)
            fire(out_copies2(c + 1, s1), s1[sem_out])

        drain(out_copies2(nch2 - 2, sets[0]), sets[0][sem_out])
        drain(out_copies2(nch2 - 1, sets[1]), sets[1][sem_out])

    outp, outn = run2(posf, negf, betaf, dp, dn)
    return outp.reshape(B, L, 1), outn.reshape(B, L, 1)


# speedup vs baseline: 1.7296x; 1.0592x over previous
import dataclasses
import functools

import jax
import jax.numpy as jnp
from jax import lax
from jax.experimental import pallas as pl
from jax.experimental.pallas import tpu as pltpu
from jax.experimental.pallas import tpu_sc as plsc

_NW = 32
_D = 64
_G = 4
_SPLITS = ((0, 104), (104, 96))


def _clip_scale(ss):
    i = plsc.bitcast(ss, jnp.int32)
    i = jnp.int32(0x5F3759DF) - (i >> 1)
    y = plsc.bitcast(i, jnp.float32)
    for _ in range(3):
        y = y * (1.5 - 0.5 * ss * y * y)
    return jnp.where(ss > 1.0, y, jnp.float32(1.0))


def kernel(uid, seq, pos, neg, user_embs, item_embs, item_beta, trans):
    B, L = seq.shape
    b_per_w = B // _NW
    nch = b_per_w // _G
    W = _G * L

    seqf = seq.reshape(-1)
    posf = pos.reshape(-1)
    negf = neg.reshape(-1)
    betaf = item_beta.reshape(-1)
    uid2 = uid.reshape(B // _G, _G)

    mesh = plsc.VectorSubcoreMesh(core_axis_name="c", subcore_axis_name="s")
    out_sds = jax.ShapeDtypeStruct((B * L,), jnp.float32)
    cp = pltpu.CompilerParams()
    for _f, _v in (("needs_layout_passes", False),
                   ("use_tc_tiling_on_sc", False)):
        if _f in pltpu.CompilerParams.__dataclass_fields__:
            cp = dataclasses.replace(cp, **{_f: _v})

    def _buf_set():
        return [
            pltpu.VMEM((W,), jnp.int32),
            pltpu.VMEM((W,), jnp.int32),
            pltpu.VMEM((W,), jnp.int32),
            pltpu.VMEM((_G, _D), jnp.float32),
            pltpu.VMEM((W, _D), jnp.float32),
            pltpu.VMEM((W, _D), jnp.float32),
            pltpu.VMEM((W, _D), jnp.float32),
            pltpu.VMEM((W,), jnp.float32),
            pltpu.VMEM((W,), jnp.float32),
            pltpu.VMEM((W,), jnp.float32),
            pltpu.VMEM((W,), jnp.float32),
            pltpu.SemaphoreType.DMA,
            pltpu.SemaphoreType.DMA,
            pltpu.SemaphoreType.DMA,
        ]

    @functools.partial(
        pl.kernel,
        mesh=mesh,
        compiler_params=cp,
        out_type=[out_sds, out_sds],
        scratch_types=[
            pltpu.VMEM((nch, _G), jnp.int32),
            pltpu.VMEM((_D,), jnp.float32),
            pltpu.VMEM((_G, _D), jnp.float32),
        ] + _buf_set() + _buf_set(),
    )
    def run(uid_hbm, seq_hbm, pos_hbm, neg_hbm, user_hbm, item_hbm, beta_hbm,
            trans_hbm, outp_hbm, outn_hbm, uid_v, trans_v, arows, *bufs):
        nb = len(_buf_set())
        sets = (bufs[:nb], bufs[nb:])
        wid = lax.axis_index("s") * 2 + lax.axis_index("c")
        tb = wid * b_per_w

        pltpu.sync_copy(uid_hbm.at[pl.ds(wid * nch, nch), :], uid_v)
        pltpu.sync_copy(trans_hbm, trans_v)

        def idx_copies(c, S):
            seqi, posi, negi = S[0], S[1], S[2]
            pb = tb * L + c * W
            return [
                (seq_hbm.at[pl.ds(pb, W)], seqi),
                (pos_hbm.at[pl.ds(pb, W)], posi),
                (neg_hbm.at[pl.ds(pb, W)], negi),
            ]

        def gather_copies(c, S):
            seqi, posi, negi, urows = S[0], S[1], S[2], S[3]
            srows, prows, nrows, pbeta, nbeta = S[4], S[5], S[6], S[7], S[8]
            out = [(user_hbm.at[uid_v.at[c]], urows)]
            for (o, n) in _SPLITS:
                sl = pl.ds(o, n)
                out += [
                    (item_hbm.at[seqi.at[sl]], srows.at[sl]),
                    (item_hbm.at[posi.at[sl]], prows.at[sl]),
                    (item_hbm.at[negi.at[sl]], nrows.at[sl]),
                    (beta_hbm.at[posi.at[sl]], pbeta.at[sl]),
                    (beta_hbm.at[negi.at[sl]], nbeta.at[sl]),
                ]
            return out

        def out_copies(c, S):
            pb = tb * L + c * W
            return [(S[9], outp_hbm.at[pl.ds(pb, W)]),
                    (S[10], outn_hbm.at[pl.ds(pb, W)])]

        def fire(pairs, sem):
            for s, d in pairs:
                pltpu.async_copy(s, d, sem)

        def drain(pairs, sem):
            for s, d in pairs:
                pltpu.make_async_copy(s, d, sem).wait()

        lanes = lax.iota(jnp.int32, 16)
        zeros = jnp.zeros((16,), jnp.float32)

        def compute(c, S):
            urows = S[3]
            srows, prows, nrows, pbeta, nbeta = S[4], S[5], S[6], S[7], S[8]
            outp_v, outn_v = S[9], S[10]

            ssu_vec = zeros
            for g in range(_G):
                acc = None
                for k in range(4):
                    u = urows[g, pl.ds(16 * k, 16)]
                    acc = u * u if acc is None else acc + u * u
                ssu_vec = jnp.where(lanes == g, jnp.sum(acc), ssu_vec)
            scu_vec = _clip_scale(ssu_vec)
            for g in range(_G):
                scu = scu_vec[g]
                for k in range(4):
                    arows[g, pl.ds(16 * k, 16)] = (
                        urows[g, pl.ds(16 * k, 16)] * scu
                        + trans_v[pl.ds(16 * k, 16)])

            def _emit_group(rows, a_of, A_of, store):
                SSs = SSp = SSn = ASv = APv = ANv = SPv = SNv = zeros
                Av = zeros
                for j, r in enumerate(rows):
                    s = [srows[r, pl.ds(16 * k, 16)] for k in range(4)]
                    p = [prows[r, pl.ds(16 * k, 16)] for k in range(4)]
                    n = [nrows[r, pl.ds(16 * k, 16)] for k in range(4)]
                    a = a_of(j)
                    m = lanes == j

                    def dot(x, y):
                        acc = x[0] * y[0]
                        for k in range(1, 4):
                            acc = acc + x[k] * y[k]
                        return jnp.sum(acc)

                    SSs = jnp.where(m, dot(s, s), SSs)
                    SSp = jnp.where(m, dot(p, p), SSp)
                    SSn = jnp.where(m, dot(n, n), SSn)
                    ASv = jnp.where(m, dot(a, s), ASv)
                    APv = jnp.where(m, dot(a, p), APv)
                    ANv = jnp.where(m, dot(a, n), ANv)
                    SPv = jnp.where(m, dot(s, p), SPv)
                    SNv = jnp.where(m, dot(s, n), SNv)
                    if A_of is None:
                        Av = jnp.where(m, dot(a, a), Av)
                if A_of is not None:
                    Av = A_of
                al = _clip_scale(SSs)
                be = _clip_scale(SSp)
                ga = _clip_scale(SSn)
                base = Av + al * al * SSs + 2.0 * al * ASv
                distp = base + be * be * SSp - 2.0 * (be * APv + al * be * SPv)
                distn = base + ga * ga * SSn - 2.0 * (ga * ANv + al * ga * SNv)
                store(distp, distn)

            @pl.loop(0, _G)
            def _b(g):
                a = [arows[g, pl.ds(16 * k, 16)] for k in range(4)]
                accA = a[0] * a[0]
                for k in range(1, 4):
                    accA = accA + a[k] * a[k]
                A = jnp.sum(accA)

                @pl.loop(0, 3)
                def _tl(tl):
                    r0 = g * L + 16 * tl

                    def store(distp, distn):
                        outp_v[pl.ds(r0, 16)] = pbeta[pl.ds(r0, 16)] - distp
                        outn_v[pl.ds(r0, 16)] = nbeta[pl.ds(r0, 16)] - distn

                    _emit_group([r0 + j for j in range(16)],
                                lambda j: a, A, store)

            lrows = [(j // 2) * L + 48 + (j % 2) if j < 2 * _G else 0
                     for j in range(16)]
            lmask = lanes < 2 * _G
            lidx = jnp.where(lmask, (lanes >> 1) * L + 48 + (lanes & 1), 0)

            def lstore(distp, distn):
                pb = plsc.load_gather(pbeta, [lidx])
                nb_ = plsc.load_gather(nbeta, [lidx])
                plsc.store_scatter(outp_v, [lidx], pb - distp, mask=lmask)
                plsc.store_scatter(outn_v, [lidx], nb_ - distn, mask=lmask)

            _emit_group(lrows,
                        lambda j: [arows[(j // 2) % _G, pl.ds(16 * k, 16)]
                                   for k in range(4)],
                        None, lstore)

        for s, d in idx_copies(0, sets[0]):
            pltpu.async_copy(s, d, sets[0][11]).wait()
        fire(gather_copies(0, sets[0]), sets[0][12])
        fire(idx_copies(1, sets[1]), sets[1][11])

        @pl.loop(0, nch, step=2)
        def _body(c):
            s0, s1 = sets
            drain(idx_copies(c + 1, s1), s1[11])
            fire(gather_copies(c + 1, s1), s1[12])
            drain(gather_copies(c, s0), s0[12])
            @pl.when(c + 2 < nch)
            def _():
                fire(idx_copies(c + 2, s0), s0[11])
            @pl.when(c >= 2)
            def _():
                drain(out_copies(c - 2, s0), s0[13])
            compute(c, s0)
            fire(out_copies(c, s0), s0[13])
            @pl.when(c + 2 < nch)
            def _():
                drain(idx_copies(c + 2, s0), s0[11])
                fire(gather_copies(c + 2, s0), s0[12])
            drain(gather_copies(c + 1, s1), s1[12])
            @pl.when(c + 2 < nch)
            def _():
                fire(idx_copies(c + 3, s1), s1[11])
            @pl.when(c >= 2)
            def _():
                drain(out_copies(c - 1, s1), s1[13])
            compute(c + 1, s1)
            fire(out_copies(c + 1, s1), s1[13])

        drain(out_copies(nch - 2, sets[0]), sets[0][13])
        drain(out_copies(nch - 1, sets[1]), sets[1][13])

    outp, outn = run(uid2, seqf, posf, negf, user_embs, item_embs, betaf,
                     trans)
    return outp.reshape(B, L, 1), outn.reshape(B, L, 1)
